# baseline (speedup 1.0000x reference)
import jax
import jax.numpy as jnp
from jax.experimental import pallas as pl
from jax.experimental.pallas import tpu as pltpu
from jax.experimental.pallas import tpu_sc as plsc

_BBLK = 8
_NCHUNK = 2


def _sc_gather_flat(idx, table):
    nb, seq = idx.shape
    _, embed = table.shape
    rows_per_step = _BBLK * seq

    mesh = plsc.VectorSubcoreMesh(
        core_axis_name="core", subcore_axis_name="subcore"
    )

    @pl.kernel(
        out_type=jax.ShapeDtypeStruct((nb * seq, embed), table.dtype),
        mesh=mesh,
        scratch_types=[pltpu.SemaphoreType.DMA],
    )
    def _gather(tab_hbm, idx_hbm, out_hbm, sem):
        def body(i_vmem, o_vmem):
            copies = [
                pltpu.async_copy(
                    tab_hbm.at[i_vmem.at[j]],
                    o_vmem.at[pl.ds(j * seq, seq), :],
                    sem,
                )
                for j in range(_BBLK)
            ]
            for c in copies:
                c.wait()

        pltpu.emit_pipeline(
            body,
            grid=(nb // _BBLK,),
            in_specs=[
                pl.BlockSpec((_BBLK, seq), index_map=lambda i: (i, 0))
            ],
            out_specs=[
                pl.BlockSpec(
                    (rows_per_step, embed), index_map=lambda i: (i, 0)
                )
            ],
            core_axis_name=("core", "subcore"),
            dimension_semantics=(pltpu.PARALLEL,),
        )(idx_hbm, out_hbm)

    return _gather(table, idx)


def _tc_relayout(y, state, batch, seq, embed, nb, off):
    rows_per_step = _BBLK * seq

    def fn(y_ref, *rest):
        o_ref = rest[-1]
        for j in range(_BBLK):
            o_ref[j] = y_ref[pl.ds(j * seq, seq), :]

    out_spec = pl.BlockSpec(
        (_BBLK, seq, embed), lambda i: (i + off // _BBLK, 0, 0)
    )
    in_specs = [pl.BlockSpec((rows_per_step, embed), lambda i: (i, 0))]
    args = [y]
    kwargs = {}
    if state is not None:
        in_specs.append(pl.BlockSpec(memory_space=pl.ANY))
        args.append(state)
        kwargs["input_output_aliases"] = {1: 0}

    return pl.pallas_call(
        fn,
        grid=(nb // _BBLK,),
        in_specs=in_specs,
        out_specs=out_spec,
        out_shape=jax.ShapeDtypeStruct((batch, seq, embed), y.dtype),
        **kwargs,
    )(*args)


def kernel(word_ids, table):
    batch, seq = word_ids.shape
    _, embed = table.shape
    idx = word_ids.astype(jnp.int32)
    chunk = batch // _NCHUNK

    state = None
    for k in range(_NCHUNK):
        y = _sc_gather_flat(idx[k * chunk : (k + 1) * chunk], table)
        state = _tc_relayout(y, state, batch, seq, embed, chunk, k * chunk)
    return state


# speedup vs baseline: 1.8096x; 1.8096x over previous
import jax
import jax.numpy as jnp
from jax.experimental import pallas as pl
from jax.experimental.pallas import tpu as pltpu
from jax.experimental.pallas import tpu_sc as plsc

_BBLK = 8
_SEQ_PAD = 56
_NCHUNK = 2
_TCBLK = 64


def _sc_gather_padded(idx, table):
    nb, seq = idx.shape
    _, embed = table.shape

    mesh = plsc.VectorSubcoreMesh(
        core_axis_name="core", subcore_axis_name="subcore"
    )

    @pl.kernel(
        out_type=jax.ShapeDtypeStruct((nb, _SEQ_PAD, embed), table.dtype),
        mesh=mesh,
        scratch_types=[pltpu.SemaphoreType.DMA],
    )
    def _gather(tab_hbm, idx_hbm, out_hbm, sem):
        def body(i_vmem, o_vmem):
            copies = [
                pltpu.async_copy(
                    tab_hbm.at[i_vmem.at[j]],
                    o_vmem.at[j, pl.ds(0, seq), :],
                    sem,
                )
                for j in range(_BBLK)
            ]
            for c in copies:
                c.wait()

        pltpu.emit_pipeline(
            body,
            grid=(nb // _BBLK,),
            in_specs=[
                pl.BlockSpec((_BBLK, seq), index_map=lambda i: (i, 0))
            ],
            out_specs=[
                pl.BlockSpec(
                    (_BBLK, _SEQ_PAD, embed), index_map=lambda i: (i, 0, 0)
                )
            ],
            core_axis_name=("core", "subcore"),
            dimension_semantics=(pltpu.PARALLEL,),
        )(idx_hbm, out_hbm)

    return _gather(table, idx)


def _tc_relayout(y, state, batch, seq, embed, nb, off):
    def fn(y_ref, *rest):
        o_ref = rest[-1]
        o_ref[...] = y_ref[:, pl.ds(0, seq), :]

    out_spec = pl.BlockSpec(
        (_TCBLK, seq, embed), lambda i: (i + off // _TCBLK, 0, 0)
    )
    in_specs = [
        pl.BlockSpec((_TCBLK, _SEQ_PAD, embed), lambda i: (i, 0, 0))
    ]
    args = [y]
    kwargs = {}
    if state is not None:
        in_specs.append(pl.BlockSpec(memory_space=pl.ANY))
        args.append(state)
        kwargs["input_output_aliases"] = {1: 0}

    return pl.pallas_call(
        fn,
        grid=(nb // _TCBLK,),
        in_specs=in_specs,
        out_specs=out_spec,
        out_shape=jax.ShapeDtypeStruct((batch, seq, embed), y.dtype),
        **kwargs,
    )(*args)


def kernel(word_ids, table):
    batch, seq = word_ids.shape
    _, embed = table.shape
    idx = word_ids.astype(jnp.int32)
    chunk = batch // _NCHUNK

    state = None
    for k in range(_NCHUNK):
        y = _sc_gather_padded(idx[k * chunk : (k + 1) * chunk], table)
        state = _tc_relayout(y, state, batch, seq, embed, chunk, k * chunk)
    return state


# speedup vs baseline: 4.2043x; 2.3233x over previous
import jax
import jax.numpy as jnp
from jax.experimental import pallas as pl
from jax.experimental.pallas import tpu as pltpu
from jax.experimental.pallas import tpu_sc as plsc

_WINDOW = 256


def kernel(word_ids, table):
    batch, seq = word_ids.shape
    _, embed = table.shape

    idx_t = word_ids.T.astype(jnp.int32)

    mesh = plsc.VectorSubcoreMesh(
        core_axis_name="core", subcore_axis_name="subcore"
    )

    @pl.kernel(
        out_type=jax.ShapeDtypeStruct((seq, batch, embed), table.dtype),
        mesh=mesh,
    )
    def _gather(tab_hbm, idx_hbm, out_hbm):
        def body(i_vmem, o_vmem):
            pltpu.sync_copy(tab_hbm.at[i_vmem.at[0]], o_vmem.at[0])

        pltpu.emit_pipeline(
            body,
            grid=(seq, batch // _WINDOW),
            in_specs=[
                pl.BlockSpec((1, _WINDOW), index_map=lambda i, j: (i, j))
            ],
            out_specs=[
                pl.BlockSpec(
                    (1, _WINDOW, embed), index_map=lambda i, j: (i, j, 0)
                )
            ],
            core_axis_name=("core", "subcore"),
            dimension_semantics=(pltpu.PARALLEL, pltpu.PARALLEL),
        )(idx_hbm, out_hbm)

    y = _gather(table, idx_t)
    return jnp.transpose(y, (1, 0, 2))


# speedup vs baseline: 4.6410x; 1.1039x over previous
import jax
from jax import lax
import jax.numpy as jnp
from jax.experimental import pallas as pl
from jax.experimental.pallas import tpu as pltpu
from jax.experimental.pallas import tpu_sc as plsc

_WINDOW = 256
_NWORKERS = 32


def kernel(word_ids, table):
    batch, seq = word_ids.shape
    _, embed = table.shape
    nwin = batch // _WINDOW
    steps = (seq * nwin) // _NWORKERS

    idx_t = word_ids.T.astype(jnp.int32)

    mesh = plsc.VectorSubcoreMesh(
        core_axis_name="core", subcore_axis_name="subcore"
    )

    @pl.kernel(
        out_type=jax.ShapeDtypeStruct((seq, batch, embed), table.dtype),
        mesh=mesh,
        scratch_types=[
            pltpu.VMEM((_WINDOW,), jnp.int32),
            pltpu.VMEM((_WINDOW,), jnp.int32),
            pltpu.VMEM((2, _WINDOW, embed), table.dtype),
            pltpu.SemaphoreType.DMA((2,)),
            pltpu.SemaphoreType.DMA((2,)),
            pltpu.SemaphoreType.DMA((2,)),
        ],
    )
    def _gather(
        tab_hbm, idx_hbm, out_hbm, idx_v0, idx_v1, rows_v, sem_i, sem_g, sem_o
    ):
        idx_bufs = (idx_v0, idx_v1)
        wid = lax.axis_index("subcore") * 2 + lax.axis_index("core")

        def win(t):
            w = wid + _NWORKERS * t
            return w // nwin, w % nwin

        def idx_copy(t, b):
            s, j = win(t)
            return pltpu.make_async_copy(
                idx_hbm.at[s, pl.ds(j * _WINDOW, _WINDOW)],
                idx_bufs[b],
                sem_i.at[b],
            )

        def gather_copy(b):
            return pltpu.make_async_copy(
                tab_hbm.at[idx_bufs[b]], rows_v.at[b], sem_g.at[b]
            )

        def out_copy(t, b):
            s, j = win(t)
            return pltpu.make_async_copy(
                rows_v.at[b],
                out_hbm.at[s, pl.ds(j * _WINDOW, _WINDOW), :],
                sem_o.at[b],
            )

        idx_copy(0, 0).start()
        idx_copy(1, 1).start()
        idx_copy(0, 0).wait()
        gather_copy(0).start()

        for t in range(steps):
            b = t % 2
            nb = (t + 1) % 2
            gather_copy(b).wait()
            out_copy(t, b).start()
            if t + 1 < steps:
                idx_copy(t + 1, nb).wait()
                if t >= 1:
                    out_copy(t - 1, nb).wait()
                gather_copy(nb).start()
                if t + 2 < steps:
                    idx_copy(t + 2, b).start()
        out_copy(steps - 2, (steps - 2) % 2).wait()
        out_copy(steps - 1, (steps - 1) % 2).wait()

    y = _gather(table, idx_t)
    return jnp.transpose(y, (1, 0, 2))
